# baseline (speedup 1.0000x reference)
import functools

import jax
import jax.numpy as jnp
from jax import lax
from jax.experimental import pallas as pl
from jax.experimental.pallas import tpu as pltpu

_TOPK = 16
_NCHUNK = 8


def _body(fv_ref, ctx_ref, wq_ref, wk_ref, wv_ref, out_ref, idx_ref, vals_ref,
          obufs, sems):
    fv = fv_ref[0]
    ctx = ctx_ref[0]
    n, d = fv.shape
    m = ctx.shape[0]
    nb = pl.num_programs(0)

    q = jnp.dot(fv, wq_ref[...], preferred_element_type=jnp.float32)
    kk = jnp.dot(ctx, wk_ref[...], preferred_element_type=jnp.float32)
    vv = jnp.dot(ctx, wv_ref[...], preferred_element_type=jnp.float32)
    kk = kk - jnp.max(kk, axis=0, keepdims=True)
    ek = jnp.exp(kk)
    kk = ek / jnp.sum(ek, axis=0, keepdims=True)
    lam = lax.dot_general(kk, vv, (((0,), (0,)), ((), ())),
                          preferred_element_type=jnp.float32)
    w = jnp.dot(q, lam, preferred_element_type=jnp.float32)

    iota_f = lax.broadcasted_iota(jnp.int32, (n, m), 1).astype(jnp.float32)
    for t in range(_TOPK):
        mx = jnp.max(w, axis=1, keepdims=True)
        am = jnp.min(jnp.where(w == mx, iota_f, float(m)), axis=1,
                     keepdims=True)
        idx_ref[:, t] = am[:, 0]
        vals_ref[:, t] = mx[:, 0]
        w = jnp.where(iota_f == am, -jnp.inf, w)

    vals = vals_ref[...]
    sm = jnp.exp(vals - jnp.max(vals, axis=1, keepdims=True))
    sm = sm / jnp.sum(sm, axis=1, keepdims=True)
    sm_b = sm.astype(jnp.bfloat16)
    idx_b = idx_ref[...].astype(jnp.bfloat16)
    ctx_b = ctx.astype(jnp.bfloat16)

    b = pl.program_id(0)
    nh = n // _NCHUNK
    iota3 = lax.broadcasted_iota(jnp.int32, (nh, _TOPK, m), 2).astype(
        jnp.bfloat16)
    for h in range(_NCHUNK):
        lo, hi = h * nh, (h + 1) * nh
        sc = jnp.where(iota3 == idx_b[lo:hi, :][:, :, None],
                       sm_b[lo:hi, :][:, :, None],
                       jnp.bfloat16(0.0))
        g = jnp.dot(sc.reshape(nh * _TOPK, m), ctx_b,
                    preferred_element_type=jnp.float32)
        outh = (g.reshape(nh, _TOPK, d) * fv[lo:hi, :][:, None, :]
                ).reshape(nh * _TOPK, d)

        @pl.when(b > 0)
        def _(h=h):
            pltpu.make_async_copy(
                obufs[h],
                out_ref.at[b - 1, pl.ds(h * nh * _TOPK, nh * _TOPK), :],
                sems.at[h]).wait()

        obufs[h][...] = outh
        pltpu.make_async_copy(
            obufs[h],
            out_ref.at[b, pl.ds(h * nh * _TOPK, nh * _TOPK), :],
            sems.at[h]).start()

    @pl.when(b == nb - 1)
    def _():
        for h in range(_NCHUNK):
            pltpu.make_async_copy(
                obufs[h],
                out_ref.at[b, pl.ds(h * nh * _TOPK, nh * _TOPK), :],
                sems.at[h]).wait()


@jax.jit
def kernel(featureVec, contextVec, Wq, Wk, Wv):
    b, n, d = featureVec.shape
    m = contextVec.shape[1]
    nh = n // _NCHUNK

    def body(fv_ref, ctx_ref, wq_ref, wk_ref, wv_ref, out_ref,
             idx_ref, vals_ref, *rest):
        obufs = rest[:_NCHUNK]
        sems = rest[_NCHUNK]
        _body(fv_ref, ctx_ref, wq_ref, wk_ref, wv_ref, out_ref,
              idx_ref, vals_ref, obufs, sems)

    return pl.pallas_call(
        body,
        grid=(b,),
        in_specs=[
            pl.BlockSpec((1, n, d), lambda i: (i, 0, 0)),
            pl.BlockSpec((1, m, d), lambda i: (i, 0, 0)),
            pl.BlockSpec((d, d), lambda i: (0, 0)),
            pl.BlockSpec((d, d), lambda i: (0, 0)),
            pl.BlockSpec((d, m), lambda i: (0, 0)),
        ],
        out_specs=pl.BlockSpec(memory_space=pl.ANY),
        out_shape=jax.ShapeDtypeStruct((b, n * _TOPK, d), jnp.float32),
        scratch_shapes=[
            pltpu.VMEM((n, _TOPK), jnp.float32),
            pltpu.VMEM((n, _TOPK), jnp.float32),
        ] + [pltpu.VMEM((nh * _TOPK, d), jnp.float32)
             for _ in range(_NCHUNK)]
        + [pltpu.SemaphoreType.DMA((_NCHUNK,))],
    )(featureVec, contextVec, Wq, Wk, Wv)


# speedup vs baseline: 1.0563x; 1.0563x over previous
import functools

import jax
import jax.numpy as jnp
from jax import lax
from jax.experimental import pallas as pl
from jax.experimental.pallas import tpu as pltpu

_TOPK = 16
_NCHUNK = 4


def _body(fv_ref, ctx_ref, wq_ref, wk_ref, wv_ref, out_ref, idx_ref, vals_ref,
          obufs, sems):
    fv = fv_ref[0]
    ctx = ctx_ref[0]
    n, d = fv.shape
    m = ctx.shape[0]
    nb = pl.num_programs(0)

    q = jnp.dot(fv, wq_ref[...], preferred_element_type=jnp.float32)
    kk = jnp.dot(ctx, wk_ref[...], preferred_element_type=jnp.float32)
    vv = jnp.dot(ctx, wv_ref[...], preferred_element_type=jnp.float32)
    kk = kk - jnp.max(kk, axis=0, keepdims=True)
    ek = jnp.exp(kk)
    kk = ek / jnp.sum(ek, axis=0, keepdims=True)
    lam = lax.dot_general(kk, vv, (((0,), (0,)), ((), ())),
                          preferred_element_type=jnp.float32)
    w = jnp.dot(q, lam, preferred_element_type=jnp.float32)

    iota_f = lax.broadcasted_iota(jnp.int32, (n, m), 1).astype(jnp.float32)
    for t in range(_TOPK):
        mx = jnp.max(w, axis=1, keepdims=True)
        am = jnp.min(jnp.where(w == mx, iota_f, float(m)), axis=1,
                     keepdims=True)
        idx_ref[:, t] = am[:, 0]
        vals_ref[:, t] = mx[:, 0]
        w = jnp.where(iota_f == am, -jnp.inf, w)

    vals = vals_ref[...]
    sm = jnp.exp(vals - jnp.max(vals, axis=1, keepdims=True))
    sm = sm / jnp.sum(sm, axis=1, keepdims=True)
    sm_b = sm.astype(jnp.bfloat16)
    idx_b = idx_ref[...].astype(jnp.bfloat16)
    ctx_b = ctx.astype(jnp.bfloat16)

    b = pl.program_id(0)
    nh = n // _NCHUNK
    iota3 = lax.broadcasted_iota(jnp.int32, (nh, _TOPK, m), 2).astype(
        jnp.bfloat16)
    for h in range(_NCHUNK):
        lo, hi = h * nh, (h + 1) * nh
        sc = jnp.where(iota3 == idx_b[lo:hi, :][:, :, None],
                       sm_b[lo:hi, :][:, :, None],
                       jnp.bfloat16(0.0))
        g = jnp.dot(sc.reshape(nh * _TOPK, m), ctx_b,
                    preferred_element_type=jnp.float32)
        outh = (g.reshape(nh, _TOPK, d) * fv[lo:hi, :][:, None, :]
                ).reshape(nh * _TOPK, d)

        @pl.when(b > 0)
        def _(h=h):
            pltpu.make_async_copy(
                obufs[h],
                out_ref.at[b - 1, pl.ds(h * nh * _TOPK, nh * _TOPK), :],
                sems.at[h]).wait()

        obufs[h][...] = outh
        pltpu.make_async_copy(
            obufs[h],
            out_ref.at[b, pl.ds(h * nh * _TOPK, nh * _TOPK), :],
            sems.at[h]).start()

    @pl.when(b == nb - 1)
    def _():
        for h in range(_NCHUNK):
            pltpu.make_async_copy(
                obufs[h],
                out_ref.at[b, pl.ds(h * nh * _TOPK, nh * _TOPK), :],
                sems.at[h]).wait()


@jax.jit
def kernel(featureVec, contextVec, Wq, Wk, Wv):
    b, n, d = featureVec.shape
    m = contextVec.shape[1]
    nh = n // _NCHUNK

    def body(fv_ref, ctx_ref, wq_ref, wk_ref, wv_ref, out_ref,
             idx_ref, vals_ref, *rest):
        obufs = rest[:_NCHUNK]
        sems = rest[_NCHUNK]
        _body(fv_ref, ctx_ref, wq_ref, wk_ref, wv_ref, out_ref,
              idx_ref, vals_ref, obufs, sems)

    return pl.pallas_call(
        body,
        grid=(b,),
        in_specs=[
            pl.BlockSpec((1, n, d), lambda i: (i, 0, 0)),
            pl.BlockSpec((1, m, d), lambda i: (i, 0, 0)),
            pl.BlockSpec((d, d), lambda i: (0, 0)),
            pl.BlockSpec((d, d), lambda i: (0, 0)),
            pl.BlockSpec((d, m), lambda i: (0, 0)),
        ],
        out_specs=pl.BlockSpec(memory_space=pl.ANY),
        out_shape=jax.ShapeDtypeStruct((b, n * _TOPK, d), jnp.float32),
        scratch_shapes=[
            pltpu.VMEM((n, _TOPK), jnp.float32),
            pltpu.VMEM((n, _TOPK), jnp.float32),
        ] + [pltpu.VMEM((nh * _TOPK, d), jnp.float32)
             for _ in range(_NCHUNK)]
        + [pltpu.SemaphoreType.DMA((_NCHUNK,))],
    )(featureVec, contextVec, Wq, Wk, Wv)
